# baseline (speedup 1.0000x reference)
import functools

import jax
import jax.numpy as jnp
from jax import lax
from jax.experimental import pallas as pl
from jax.experimental.pallas import tpu as pltpu
from jax.experimental.pallas import tpu_sc as plsc

H = 32
W = 32
NH = 16
M = 1024
N = 1024
TH = 2 * H - 1
TW = 2 * W - 1
TSZ = TH * TW
TSTRIDE = 3976

NC = 2
NS = 16
L = 16
NW = NC * NS
HG = 4
RG = NW // (NH // HG)
ROWS = M // RG
RB = 4
RSUB = 4


def _body(qc_h, kc_h, tab_h, out_h, tab_v, kc_v, qc_v,
          buf0, buf1, buf2, buf3, sem0, sem1, sem2, sem3):
    wid = lax.axis_index("s") * NC + lax.axis_index("c")
    hgroup = wid // RG
    rgroup = wid % RG
    rbase = rgroup * ROWS

    cp0 = pltpu.async_copy(
        tab_h.at[pl.ds(hgroup * (HG * TSTRIDE), HG * TSTRIDE)], tab_v, sem0)
    cp1 = pltpu.async_copy(kc_h, kc_v, sem0)
    cp2 = pltpu.async_copy(qc_h.at[pl.ds(rbase * L, ROWS * L)], qc_v, sem0)
    cp0.wait()
    cp1.wait()
    cp2.wait()

    bufs = (buf0, buf1, buf2, buf3)
    sems = (sem0, sem1, sem2, sem3)

    def iteration(i, carry):
        for b in range(4):
            buf = bufs[b]
            sem = sems[b]
            g0 = i * (4 * RB) + b * RB

            @pl.when(i >= 1)
            def _():
                pltpu.make_async_copy(
                    buf, out_h.at[pl.ds(0, HG), pl.ds(0, RB)], sem).wait()

            for sub in range(RB // RSUB):
                qcs = [qc_v[pl.ds((g0 + sub * RSUB + rr) * L, L)]
                       for rr in range(RSUB)]

                @plsc.parallel_loop(0, N, step=L, unroll=2)
                def chunk(off):
                    kc = kc_v[pl.ds(off, L)]
                    for rr in range(RSUB):
                        flat = qcs[rr] - kc
                        for hh in range(HG):
                            vals = plsc.load_gather(tab_v, [flat + hh * TSTRIDE])
                            buf[hh, sub * RSUB + rr, pl.ds(off, L)] = vals

            pltpu.async_copy(
                buf,
                out_h.at[pl.ds(hgroup * HG, HG), pl.ds(rbase + g0, RB)],
                sem)
        return carry

    lax.fori_loop(0, ROWS // (4 * RB), iteration, 0)

    for b in range(4):
        pltpu.make_async_copy(
            bufs[b], out_h.at[pl.ds(0, HG), pl.ds(0, RB)], sems[b]).wait()


_sc_call = functools.partial(
    pl.kernel,
    out_type=jax.ShapeDtypeStruct((NH, M, N), jnp.float32),
    mesh=plsc.VectorSubcoreMesh(core_axis_name="c", subcore_axis_name="s"),
    compiler_params=pltpu.CompilerParams(needs_layout_passes=False),
    scratch_types=[
        pltpu.VMEM((HG * TSTRIDE,), jnp.float32),
        pltpu.VMEM((N,), jnp.int32),
        pltpu.VMEM((ROWS * L,), jnp.int32),
        pltpu.VMEM((HG, RB, N), jnp.float32),
        pltpu.VMEM((HG, RB, N), jnp.float32),
        pltpu.VMEM((HG, RB, N), jnp.float32),
        pltpu.VMEM((HG, RB, N), jnp.float32),
        pltpu.SemaphoreType.DMA,
        pltpu.SemaphoreType.DMA,
        pltpu.SemaphoreType.DMA,
        pltpu.SemaphoreType.DMA,
    ],
)(_body)


@jax.jit
def kernel(q_coords, k_coords, bias):
    qc = q_coords[:, 0].astype(jnp.int32) * TW + q_coords[:, 1].astype(jnp.int32)
    qc = qc + ((H - 1) * TW + (W - 1))
    qc = jnp.broadcast_to(qc[:, None], (M, L)).reshape(M * L)
    kc = k_coords[:, 0].astype(jnp.int32) * TW + k_coords[:, 1].astype(jnp.int32)
    tab = jnp.pad(bias.reshape(NH, TSZ).astype(jnp.float32),
                  ((0, 0), (0, TSTRIDE - TSZ))).reshape(NH * TSTRIDE)
    return _sc_call(qc, kc, tab)


# speedup vs baseline: 1.0145x; 1.0145x over previous
import functools

import jax
import jax.numpy as jnp
from jax import lax
from jax.experimental import pallas as pl
from jax.experimental.pallas import tpu as pltpu
from jax.experimental.pallas import tpu_sc as plsc

H = 32
W = 32
NH = 16
M = 1024
N = 1024
TH = 2 * H - 1
TW = 2 * W - 1
TSZ = TH * TW
TSTRIDE = 3976

NC = 2
NS = 16
L = 16
NW = NC * NS
HG = 4
RG = NW // (NH // HG)
ROWS = M // RG
RB = 8
RSUB = 4


def _body(qc_h, kc_h, tab_h, out_h, tab_v, kc_v, qc_v, buf0, buf1, sem0, sem1):
    wid = lax.axis_index("s") * NC + lax.axis_index("c")
    hgroup = wid // RG
    rgroup = wid % RG
    rbase = rgroup * ROWS

    cp0 = pltpu.async_copy(
        tab_h.at[pl.ds(hgroup * (HG * TSTRIDE), HG * TSTRIDE)], tab_v, sem0)
    cp1 = pltpu.async_copy(kc_h, kc_v, sem0)
    cp2 = pltpu.async_copy(qc_h.at[pl.ds(rbase * L, ROWS * L)], qc_v, sem0)
    cp0.wait()
    cp1.wait()
    cp2.wait()

    bufs = (buf0, buf1)
    sems = (sem0, sem1)

    def iteration(i, carry):
        for b in range(2):
            buf = bufs[b]
            sem = sems[b]
            g0 = i * (2 * RB) + b * RB

            @pl.when(i >= 1)
            def _():
                pltpu.make_async_copy(
                    buf, out_h.at[pl.ds(0, HG), pl.ds(0, RB)], sem).wait()

            for sub in range(RB // RSUB):
                qcs = [qc_v[pl.ds((g0 + sub * RSUB + rr) * L, L)]
                       for rr in range(RSUB)]

                @plsc.parallel_loop(0, N, step=L, unroll=2)
                def chunk(off):
                    kc = kc_v[pl.ds(off, L)]
                    for rr in range(RSUB):
                        flat = qcs[rr] - kc
                        for hh in range(HG):
                            vals = plsc.load_gather(tab_v, [flat + hh * TSTRIDE])
                            buf[hh, sub * RSUB + rr, pl.ds(off, L)] = vals

            pltpu.async_copy(
                buf,
                out_h.at[pl.ds(hgroup * HG, HG), pl.ds(rbase + g0, RB)],
                sem)
        return carry

    lax.fori_loop(0, ROWS // (2 * RB), iteration, 0)

    for b in range(2):
        pltpu.make_async_copy(
            bufs[b], out_h.at[pl.ds(0, HG), pl.ds(0, RB)], sems[b]).wait()


_sc_call = functools.partial(
    pl.kernel,
    out_type=jax.ShapeDtypeStruct((NH, M, N), jnp.float32),
    mesh=plsc.VectorSubcoreMesh(core_axis_name="c", subcore_axis_name="s"),
    compiler_params=pltpu.CompilerParams(needs_layout_passes=False),
    scratch_types=[
        pltpu.VMEM((HG * TSTRIDE,), jnp.float32),
        pltpu.VMEM((N,), jnp.int32),
        pltpu.VMEM((ROWS * L,), jnp.int32),
        pltpu.VMEM((HG, RB, N), jnp.float32),
        pltpu.VMEM((HG, RB, N), jnp.float32),
        pltpu.SemaphoreType.DMA,
        pltpu.SemaphoreType.DMA,
    ],
)(_body)


@jax.jit
def kernel(q_coords, k_coords, bias):
    qc = q_coords[:, 0].astype(jnp.int32) * TW + q_coords[:, 1].astype(jnp.int32)
    qc = qc + ((H - 1) * TW + (W - 1))
    qc = jnp.broadcast_to(qc[:, None], (M, L)).reshape(M * L)
    kc = k_coords[:, 0].astype(jnp.int32) * TW + k_coords[:, 1].astype(jnp.int32)
    tab = jnp.pad(bias.reshape(NH, TSZ).astype(jnp.float32),
                  ((0, 0), (0, TSTRIDE - TSZ))).reshape(NH * TSTRIDE)
    return _sc_call(qc, kc, tab)
